# baseline (speedup 1.0000x reference)
import functools

import jax
import jax.numpy as jnp
from jax import lax
from jax.experimental import pallas as pl
from jax.experimental.pallas import tpu as pltpu
from jax.experimental.pallas import tpu_sc as plsc

N = 10000
NP = 10240
E = 160000
R = 3
DIN = 128
DH = 128
DOUT = 16

NC = 2
NS = 16
NW = NC * NS

CHUNK = 128
KB = 4
NCHUNK = 120
EPW = NCHUNK * CHUNK
EEP = NW * EPW
ER = EEP // CHUNK
DB = 3 * NP
ZSL = DB // NS
RPT = NP // NS


def _mesh():
    return plsc.VectorSubcoreMesh(core_axis_name="c", subcore_axis_name="s")


def _deg(dst3_2d):
    @functools.partial(
        pl.kernel,
        out_type=jax.ShapeDtypeStruct((NC, DB), jnp.float32),
        mesh=_mesh(),
        scratch_types=[
            pltpu.VMEM((KB, CHUNK), jnp.int32),
            pltpu.VMEM((CHUNK,), jnp.float32),
            pltpu.VMEM((ZSL,), jnp.float32),
            pltpu.VMEM_SHARED((DB,), jnp.float32),
        ],
    )
    def deg_k(dst3_hbm, out_hbm, idx_v, ones_v, zbuf_v, acc_sh):
        cid = lax.axis_index("c")
        sid = lax.axis_index("s")
        wid = cid * NS + sid
        for j in range(CHUNK // 16):
            ones_v[pl.ds(16 * j, 16)] = jnp.ones((16,), jnp.float32)

        def zb(i, carry):
            zbuf_v[pl.ds(i * 16, 16)] = jnp.zeros((16,), jnp.float32)
            return carry

        lax.fori_loop(0, ZSL // 16, zb, 0)
        pltpu.sync_copy(zbuf_v, acc_sh.at[pl.ds(sid * ZSL, ZSL)])
        plsc.subcore_barrier()

        def blk(b, carry):
            row0 = wid * NCHUNK + b * KB
            pltpu.sync_copy(dst3_hbm.at[pl.ds(row0, KB)], idx_v)
            for k in range(KB):
                pltpu.sync_copy(ones_v, acc_sh.at[idx_v.at[k]], add=True)
            return carry

        lax.fori_loop(0, NCHUNK // KB, blk, 0)
        plsc.subcore_barrier()
        pltpu.sync_copy(acc_sh.at[pl.ds(sid * ZSL, ZSL)],
                        out_hbm.at[cid, pl.ds(sid * ZSL, ZSL)])

    return deg_k(dst3_2d)


SW = DB // NW

def _inv(deg0, deg1):
    @functools.partial(
        pl.kernel,
        out_type=jax.ShapeDtypeStruct((DB,), jnp.float32),
        mesh=_mesh(),
        scratch_types=[
            pltpu.VMEM((SW,), jnp.float32),
            pltpu.VMEM((SW,), jnp.float32),
            pltpu.VMEM((SW,), jnp.float32),
        ],
    )
    def inv_k(d0_hbm, d1_hbm, inv_hbm, d0_v, d1_v, o_v):
        cid = lax.axis_index("c")
        sid = lax.axis_index("s")
        wid = cid * NS + sid
        base = wid * SW
        pltpu.sync_copy(d0_hbm.at[pl.ds(base, SW)], d0_v)
        pltpu.sync_copy(d1_hbm.at[pl.ds(base, SW)], d1_v)

        def gg(i, carry):
            d = d0_v[pl.ds(i * 16, 16)] + d1_v[pl.ds(i * 16, 16)]
            inv = 1.0 / jnp.maximum(d, 1.0)
            b = base + i * 16 + lax.iota(jnp.int32, 16)
            o_v[pl.ds(i * 16, 16)] = jnp.where(b % NP < N, inv, 0.0)
            return carry

        lax.fori_loop(0, SW // 16, gg, 0)
        pltpu.sync_copy(o_v, inv_hbm.at[pl.ds(base, SW)])

    return inv_k(deg0, deg1)


def _edge_pass(src3, dst, dst3, inv, y3, sl, zz, d):
    @functools.partial(
        pl.kernel,
        out_type=jax.ShapeDtypeStruct((NC, NP, d), jnp.float32),
        mesh=_mesh(),
        scratch_types=[
            pltpu.VMEM((KB, CHUNK), jnp.int32),
            pltpu.VMEM((KB, CHUNK), jnp.int32),
            pltpu.VMEM((KB, CHUNK), jnp.int32),
            pltpu.VMEM((CHUNK,), jnp.float32),
            pltpu.VMEM((CHUNK,), jnp.float32),
            pltpu.VMEM((CHUNK, d), jnp.float32),
            pltpu.VMEM((CHUNK, d), jnp.float32),
            pltpu.VMEM_SHARED((NP, d), jnp.float32),
            pltpu.SemaphoreType.DMA,
            pltpu.SemaphoreType.DMA,
            pltpu.SemaphoreType.DMA,
            pltpu.SemaphoreType.DMA,
        ],
    )
    def e_k(src3_hbm, dst_hbm, dst3_hbm, inv_hbm, y3_hbm, sl_hbm, zz_hbm, out_hbm,
            sidx_v, didx_v, d3idx_v, wv_a, wv_b, rows_a, rows_b, acc_sh,
            gsem_a, gsem_b, wsem_a, wsem_b):
        cid = lax.axis_index("c")
        sid = lax.axis_index("s")
        wid = cid * NS + sid

        @pl.when(cid == 0)
        def _():
            pltpu.sync_copy(sl_hbm.at[pl.ds(sid * RPT, RPT)],
                            acc_sh.at[pl.ds(sid * RPT, RPT)])

        @pl.when(cid != 0)
        def _():
            pltpu.sync_copy(zz_hbm.at[pl.ds(sid * RPT, RPT)],
                            acc_sh.at[pl.ds(sid * RPT, RPT)])

        plsc.subcore_barrier()
        bufs = (rows_a, rows_b)
        wbufs = (wv_a, wv_b)
        gsems = (gsem_a, gsem_b)
        wsems = (wsem_a, wsem_b)

        def blk(b, carry):
            row0 = wid * NCHUNK + b * KB
            pltpu.sync_copy(src3_hbm.at[pl.ds(row0, KB)], sidx_v)
            pltpu.sync_copy(dst_hbm.at[pl.ds(row0, KB)], didx_v)
            pltpu.sync_copy(dst3_hbm.at[pl.ds(row0, KB)], d3idx_v)
            gd = [None, None]
            wd = [None, None]
            gd[0] = pltpu.async_copy(y3_hbm.at[sidx_v.at[0]], bufs[0], gsems[0])
            wd[0] = pltpu.async_copy(inv_hbm.at[d3idx_v.at[0]], wbufs[0], wsems[0])
            for k in range(KB):
                pb = k % 2
                nk = k + 1
                if nk < KB:
                    gd[nk % 2] = pltpu.async_copy(
                        y3_hbm.at[sidx_v.at[nk]], bufs[nk % 2], gsems[nk % 2])
                    wd[nk % 2] = pltpu.async_copy(
                        inv_hbm.at[d3idx_v.at[nk]], wbufs[nk % 2], wsems[nk % 2])
                gd[pb].wait()
                wd[pb].wait()
                rows_v = bufs[pb]
                wv_v = wbufs[pb]

                def rb(g2, rcarry):
                    for u in range(2):
                        g = g2 * 2 + u
                        wv16 = wv_v[pl.ds(g * 16, 16)]
                        for l in range(16):
                            i = g * 16 + l
                            ws = wv16[l]
                            for j in range(d // 16):
                                rows_v[i, pl.ds(16 * j, 16)] = (
                                    rows_v[i, pl.ds(16 * j, 16)] * ws)
                    return rcarry

                lax.fori_loop(0, CHUNK // 32, rb, 0)
                pltpu.sync_copy(rows_v, acc_sh.at[didx_v.at[k]], add=True)
            return carry

        lax.fori_loop(0, NCHUNK // KB, blk, 0)
        plsc.subcore_barrier()
        pltpu.sync_copy(acc_sh.at[pl.ds(sid * RPT, RPT)],
                        out_hbm.at[cid, pl.ds(sid * RPT, RPT)])

    return e_k(src3, dst, dst3, inv, y3, sl, zz)


def _mm(xin, W, lw, b, dout, fuse):
    BM = 512
    grid = (NP // BM,)

    def body(x_ref, w_ref, lw_ref, b_ref, y3_ref, sl_ref):
        if fuse:
            xb = jax.nn.relu(x_ref[0] + x_ref[1])
        else:
            xb = x_ref[...]
        for r in range(R):
            y3_ref[r] = jnp.dot(xb, w_ref[r], preferred_element_type=jnp.float32)
        sl_ref[...] = (jnp.dot(xb, lw_ref[...], preferred_element_type=jnp.float32)
                       + b_ref[...])

    din = xin.shape[-1]
    if fuse:
        x_spec = pl.BlockSpec((2, BM, din), lambda i: (0, i, 0))
    else:
        x_spec = pl.BlockSpec((BM, din), lambda i: (i, 0))
    return pl.pallas_call(
        body,
        grid=grid,
        in_specs=[
            x_spec,
            pl.BlockSpec((R, din, dout), lambda i: (0, 0, 0)),
            pl.BlockSpec((din, dout), lambda i: (0, 0)),
            pl.BlockSpec((1, dout), lambda i: (0, 0)),
        ],
        out_specs=[
            pl.BlockSpec((R, BM, dout), lambda i: (0, i, 0)),
            pl.BlockSpec((BM, dout), lambda i: (i, 0)),
        ],
        out_shape=[
            jax.ShapeDtypeStruct((R, NP, dout), jnp.float32),
            jax.ShapeDtypeStruct((NP, dout), jnp.float32),
        ],
    )(xin, W, lw, b)


def _final_add(acc):
    BM = 1024

    def body(a_ref, o_ref):
        o_ref[...] = (a_ref[0] + a_ref[1])[:, :DOUT]

    return pl.pallas_call(
        body,
        grid=(NP // BM,),
        in_specs=[pl.BlockSpec((2, BM, DH), lambda i: (0, i, 0))],
        out_specs=pl.BlockSpec((BM, DOUT), lambda i: (i, 0)),
        out_shape=jax.ShapeDtypeStruct((NP, DOUT), jnp.float32),
    )(acc)


def kernel(x, edge_index_r0, edge_index_r1, edge_index_r2,
           W0, loopW0, loopb0, W1, loopW1, loopb1, W2, loopW2, loopb2):
    pad = EEP - R * E
    pr = jnp.arange(pad, dtype=jnp.int32)
    e0, e1, e2 = edge_index_r0, edge_index_r1, edge_index_r2
    src3 = jnp.concatenate(
        [e0[0], e1[0] + NP, e2[0] + 2 * NP, pr % N]).astype(jnp.int32)
    dst3 = jnp.concatenate(
        [e0[1], e1[1] + NP, e2[1] + 2 * NP, N + pr % (NP - N)]).astype(jnp.int32)
    dst = jnp.concatenate([e0[1], e1[1], e2[1], pr % N]).astype(jnp.int32)

    src3_2d = src3.reshape(ER, CHUNK)
    dst_2d = dst.reshape(ER, CHUNK)
    dst3_2d = dst3.reshape(ER, CHUNK)

    degP = _deg(dst3_2d)
    inv = _inv(degP[0], degP[1])

    zz_h = jnp.zeros((NP, DH), jnp.float32)
    zz_o = jnp.zeros((NP, DOUT), jnp.float32)

    x_p = jnp.pad(x, ((0, NP - N), (0, 0)))
    y3, sl = _mm(x_p, W0, loopW0, loopb0.reshape(1, -1), DH, fuse=False)
    acc = _edge_pass(src3_2d, dst_2d, dst3_2d, inv, y3.reshape(R * NP, DH), sl, zz_h, DH)

    y3, sl = _mm(acc, W1, loopW1, loopb1.reshape(1, -1), DH, fuse=True)
    acc = _edge_pass(src3_2d, dst_2d, dst3_2d, inv, y3.reshape(R * NP, DH), sl, zz_h, DH)

    W2p = jnp.pad(W2, ((0, 0), (0, 0), (0, DH - DOUT)))
    lw2p = jnp.pad(loopW2, ((0, 0), (0, DH - DOUT)))
    b2p = jnp.pad(loopb2, (0, DH - DOUT))
    y3, sl = _mm(acc, W2p, lw2p, b2p.reshape(1, -1), DH, fuse=True)
    acc = _edge_pass(src3_2d, dst_2d, dst3_2d, inv, y3.reshape(R * NP, DH), sl, zz_h, DH)

    return _final_add(acc)[:N]


# speedup vs baseline: 1.1548x; 1.1548x over previous
import functools

import jax
import jax.numpy as jnp
from jax import lax
from jax.experimental import pallas as pl
from jax.experimental.pallas import tpu as pltpu
from jax.experimental.pallas import tpu_sc as plsc

N = 10000
NP = 10240
E = 160000
R = 3
DIN = 128
DH = 128
DOUT = 16

NC = 2
NS = 16
NW = NC * NS

CHUNK = 128
KB = 8
NCHUNK = 120
EPW = NCHUNK * CHUNK
EEP = NW * EPW
ER = EEP // CHUNK
DB = 3 * NP
ZSL = DB // NS
RPT = NP // NS


def _mesh():
    return plsc.VectorSubcoreMesh(core_axis_name="c", subcore_axis_name="s")


def _deg(dst3_2d):
    @functools.partial(
        pl.kernel,
        out_type=jax.ShapeDtypeStruct((NC, DB), jnp.float32),
        mesh=_mesh(),
        scratch_types=[
            pltpu.VMEM((KB, CHUNK), jnp.int32),
            pltpu.VMEM((CHUNK,), jnp.float32),
            pltpu.VMEM((ZSL,), jnp.float32),
            pltpu.VMEM_SHARED((DB,), jnp.float32),
        ],
    )
    def deg_k(dst3_hbm, out_hbm, idx_v, ones_v, zbuf_v, acc_sh):
        cid = lax.axis_index("c")
        sid = lax.axis_index("s")
        wid = cid * NS + sid
        for j in range(CHUNK // 16):
            ones_v[pl.ds(16 * j, 16)] = jnp.ones((16,), jnp.float32)

        def zb(i, carry):
            zbuf_v[pl.ds(i * 16, 16)] = jnp.zeros((16,), jnp.float32)
            return carry

        lax.fori_loop(0, ZSL // 16, zb, 0)
        pltpu.sync_copy(zbuf_v, acc_sh.at[pl.ds(sid * ZSL, ZSL)])
        plsc.subcore_barrier()

        def blk(b, carry):
            row0 = wid * NCHUNK + b * KB
            pltpu.sync_copy(dst3_hbm.at[pl.ds(row0, KB)], idx_v)
            for k in range(KB):
                pltpu.sync_copy(ones_v, acc_sh.at[idx_v.at[k]], add=True)
            return carry

        lax.fori_loop(0, NCHUNK // KB, blk, 0)
        plsc.subcore_barrier()
        pltpu.sync_copy(acc_sh.at[pl.ds(sid * ZSL, ZSL)],
                        out_hbm.at[cid, pl.ds(sid * ZSL, ZSL)])

    return deg_k(dst3_2d)


SW = DB // NW

def _inv(deg0, deg1):
    @functools.partial(
        pl.kernel,
        out_type=jax.ShapeDtypeStruct((DB,), jnp.float32),
        mesh=_mesh(),
        scratch_types=[
            pltpu.VMEM((SW,), jnp.float32),
            pltpu.VMEM((SW,), jnp.float32),
            pltpu.VMEM((SW,), jnp.float32),
        ],
    )
    def inv_k(d0_hbm, d1_hbm, inv_hbm, d0_v, d1_v, o_v):
        cid = lax.axis_index("c")
        sid = lax.axis_index("s")
        wid = cid * NS + sid
        base = wid * SW
        pltpu.sync_copy(d0_hbm.at[pl.ds(base, SW)], d0_v)
        pltpu.sync_copy(d1_hbm.at[pl.ds(base, SW)], d1_v)

        def gg(i, carry):
            d = d0_v[pl.ds(i * 16, 16)] + d1_v[pl.ds(i * 16, 16)]
            inv = 1.0 / jnp.maximum(d, 1.0)
            b = base + i * 16 + lax.iota(jnp.int32, 16)
            o_v[pl.ds(i * 16, 16)] = jnp.where(b % NP < N, inv, 0.0)
            return carry

        lax.fori_loop(0, SW // 16, gg, 0)
        pltpu.sync_copy(o_v, inv_hbm.at[pl.ds(base, SW)])

    return inv_k(deg0, deg1)


def _edge_pass(src3, dst, dst3, inv, y3, sl, zz, d):
    @functools.partial(
        pl.kernel,
        out_type=jax.ShapeDtypeStruct((NC, NP, d), jnp.float32),
        mesh=_mesh(),
        scratch_types=[
            pltpu.VMEM((KB, CHUNK), jnp.int32),
            pltpu.VMEM((KB, CHUNK), jnp.int32),
            pltpu.VMEM((KB, CHUNK), jnp.int32),
            pltpu.VMEM((CHUNK,), jnp.float32),
            pltpu.VMEM((CHUNK,), jnp.float32),
            pltpu.VMEM((CHUNK, d), jnp.float32),
            pltpu.VMEM((CHUNK, d), jnp.float32),
            pltpu.VMEM_SHARED((NP, d), jnp.float32),
            pltpu.SemaphoreType.DMA,
            pltpu.SemaphoreType.DMA,
            pltpu.SemaphoreType.DMA,
            pltpu.SemaphoreType.DMA,
        ],
    )
    def e_k(src3_hbm, dst_hbm, dst3_hbm, inv_hbm, y3_hbm, sl_hbm, zz_hbm, out_hbm,
            sidx_v, didx_v, d3idx_v, wv_a, wv_b, rows_a, rows_b, acc_sh,
            gsem_a, gsem_b, wsem_a, wsem_b):
        cid = lax.axis_index("c")
        sid = lax.axis_index("s")
        wid = cid * NS + sid

        @pl.when(cid == 0)
        def _():
            pltpu.sync_copy(sl_hbm.at[pl.ds(sid * RPT, RPT)],
                            acc_sh.at[pl.ds(sid * RPT, RPT)])

        @pl.when(cid != 0)
        def _():
            pltpu.sync_copy(zz_hbm.at[pl.ds(sid * RPT, RPT)],
                            acc_sh.at[pl.ds(sid * RPT, RPT)])

        plsc.subcore_barrier()
        bufs = (rows_a, rows_b)
        wbufs = (wv_a, wv_b)
        gsems = (gsem_a, gsem_b)
        wsems = (wsem_a, wsem_b)

        def blk(b, carry):
            row0 = wid * NCHUNK + b * KB
            pltpu.sync_copy(src3_hbm.at[pl.ds(row0, KB)], sidx_v)
            pltpu.sync_copy(dst_hbm.at[pl.ds(row0, KB)], didx_v)
            pltpu.sync_copy(dst3_hbm.at[pl.ds(row0, KB)], d3idx_v)
            gd = [None, None]
            wd = [None, None]
            gd[0] = pltpu.async_copy(y3_hbm.at[sidx_v.at[0]], bufs[0], gsems[0])
            wd[0] = pltpu.async_copy(inv_hbm.at[d3idx_v.at[0]], wbufs[0], wsems[0])
            for k in range(KB):
                pb = k % 2
                nk = k + 1
                if nk < KB:
                    gd[nk % 2] = pltpu.async_copy(
                        y3_hbm.at[sidx_v.at[nk]], bufs[nk % 2], gsems[nk % 2])
                    wd[nk % 2] = pltpu.async_copy(
                        inv_hbm.at[d3idx_v.at[nk]], wbufs[nk % 2], wsems[nk % 2])
                gd[pb].wait()
                wd[pb].wait()
                rows_v = bufs[pb]
                wv_v = wbufs[pb]

                def rb(g2, rcarry):
                    for u in range(2):
                        g = g2 * 2 + u
                        wv16 = wv_v[pl.ds(g * 16, 16)]
                        for l in range(16):
                            i = g * 16 + l
                            ws = wv16[l]
                            for j in range(d // 16):
                                rows_v[i, pl.ds(16 * j, 16)] = (
                                    rows_v[i, pl.ds(16 * j, 16)] * ws)
                    return rcarry

                lax.fori_loop(0, CHUNK // 32, rb, 0)
                pltpu.sync_copy(rows_v, acc_sh.at[didx_v.at[k]], add=True)
            return carry

        lax.fori_loop(0, NCHUNK // KB, blk, 0)
        plsc.subcore_barrier()
        pltpu.sync_copy(acc_sh.at[pl.ds(sid * RPT, RPT)],
                        out_hbm.at[cid, pl.ds(sid * RPT, RPT)])

    return e_k(src3, dst, dst3, inv, y3, sl, zz)


def _mm(xin, W, lw, b, dout, fuse):
    BM = 512
    grid = (NP // BM,)

    def body(x_ref, w_ref, lw_ref, b_ref, y3_ref, sl_ref):
        if fuse:
            xb = jax.nn.relu(x_ref[0] + x_ref[1])
        else:
            xb = x_ref[...]
        for r in range(R):
            y3_ref[r] = jnp.dot(xb, w_ref[r], preferred_element_type=jnp.float32)
        sl_ref[...] = (jnp.dot(xb, lw_ref[...], preferred_element_type=jnp.float32)
                       + b_ref[...])

    din = xin.shape[-1]
    if fuse:
        x_spec = pl.BlockSpec((2, BM, din), lambda i: (0, i, 0))
    else:
        x_spec = pl.BlockSpec((BM, din), lambda i: (i, 0))
    return pl.pallas_call(
        body,
        grid=grid,
        in_specs=[
            x_spec,
            pl.BlockSpec((R, din, dout), lambda i: (0, 0, 0)),
            pl.BlockSpec((din, dout), lambda i: (0, 0)),
            pl.BlockSpec((1, dout), lambda i: (0, 0)),
        ],
        out_specs=[
            pl.BlockSpec((R, BM, dout), lambda i: (0, i, 0)),
            pl.BlockSpec((BM, dout), lambda i: (i, 0)),
        ],
        out_shape=[
            jax.ShapeDtypeStruct((R, NP, dout), jnp.float32),
            jax.ShapeDtypeStruct((NP, dout), jnp.float32),
        ],
    )(xin, W, lw, b)


def _final_add(acc):
    BM = 1024

    def body(a_ref, o_ref):
        o_ref[...] = (a_ref[0] + a_ref[1])[:, :DOUT]

    return pl.pallas_call(
        body,
        grid=(NP // BM,),
        in_specs=[pl.BlockSpec((2, BM, DH), lambda i: (0, i, 0))],
        out_specs=pl.BlockSpec((BM, DOUT), lambda i: (i, 0)),
        out_shape=jax.ShapeDtypeStruct((NP, DOUT), jnp.float32),
    )(acc)


def kernel(x, edge_index_r0, edge_index_r1, edge_index_r2,
           W0, loopW0, loopb0, W1, loopW1, loopb1, W2, loopW2, loopb2):
    pad = EEP - R * E
    pr = jnp.arange(pad, dtype=jnp.int32)
    e0, e1, e2 = edge_index_r0, edge_index_r1, edge_index_r2
    src3 = jnp.concatenate(
        [e0[0], e1[0] + NP, e2[0] + 2 * NP, pr % N]).astype(jnp.int32)
    dst3 = jnp.concatenate(
        [e0[1], e1[1] + NP, e2[1] + 2 * NP, N + pr % (NP - N)]).astype(jnp.int32)
    dst = jnp.concatenate([e0[1], e1[1], e2[1], pr % N]).astype(jnp.int32)

    src3_2d = src3.reshape(ER, CHUNK)
    dst_2d = dst.reshape(ER, CHUNK)
    dst3_2d = dst3.reshape(ER, CHUNK)

    degP = _deg(dst3_2d)
    inv = _inv(degP[0], degP[1])

    zz_h = jnp.zeros((NP, DH), jnp.float32)
    zz_o = jnp.zeros((NP, DOUT), jnp.float32)

    x_p = jnp.pad(x, ((0, NP - N), (0, 0)))
    y3, sl = _mm(x_p, W0, loopW0, loopb0.reshape(1, -1), DH, fuse=False)
    acc = _edge_pass(src3_2d, dst_2d, dst3_2d, inv, y3.reshape(R * NP, DH), sl, zz_h, DH)

    y3, sl = _mm(acc, W1, loopW1, loopb1.reshape(1, -1), DH, fuse=True)
    acc = _edge_pass(src3_2d, dst_2d, dst3_2d, inv, y3.reshape(R * NP, DH), sl, zz_h, DH)

    W2p = jnp.pad(W2, ((0, 0), (0, 0), (0, DH - DOUT)))
    lw2p = jnp.pad(loopW2, ((0, 0), (0, DH - DOUT)))
    b2p = jnp.pad(loopb2, (0, DH - DOUT))
    y3, sl = _mm(acc, W2p, lw2p, b2p.reshape(1, -1), DH, fuse=True)
    acc = _edge_pass(src3_2d, dst_2d, dst3_2d, inv, y3.reshape(R * NP, DH), sl, zz_h, DH)

    return _final_add(acc)[:N]


# speedup vs baseline: 1.2821x; 1.1103x over previous
import functools

import jax
import jax.numpy as jnp
from jax import lax
from jax.experimental import pallas as pl
from jax.experimental.pallas import tpu as pltpu
from jax.experimental.pallas import tpu_sc as plsc

N = 10000
NP = 10240
E = 160000
R = 3
DIN = 128
DH = 128
DOUT = 16

NC = 2
NS = 16
NW = NC * NS

CHUNK = 128
KB = 8
NCHUNK = 120
NCHUNK_R = 40
PR_ROWS = NW * NCHUNK_R
EPW = NCHUNK * CHUNK
EEP = NW * EPW
ER = EEP // CHUNK
DB = 3 * NP
ZSL = DB // NS
RPT = NP // NS


def _mesh():
    return plsc.VectorSubcoreMesh(core_axis_name="c", subcore_axis_name="s")


def _deg(dst3_2d):
    @functools.partial(
        pl.kernel,
        out_type=jax.ShapeDtypeStruct((NC, DB), jnp.float32),
        mesh=_mesh(),
        scratch_types=[
            pltpu.VMEM((KB, CHUNK), jnp.int32),
            pltpu.VMEM((CHUNK,), jnp.float32),
            pltpu.VMEM((ZSL,), jnp.float32),
            pltpu.VMEM_SHARED((DB,), jnp.float32),
        ],
    )
    def deg_k(dst3_hbm, out_hbm, idx_v, ones_v, zbuf_v, acc_sh):
        cid = lax.axis_index("c")
        sid = lax.axis_index("s")
        wid = cid * NS + sid
        for j in range(CHUNK // 16):
            ones_v[pl.ds(16 * j, 16)] = jnp.ones((16,), jnp.float32)

        def zb(i, carry):
            zbuf_v[pl.ds(i * 16, 16)] = jnp.zeros((16,), jnp.float32)
            return carry

        lax.fori_loop(0, ZSL // 16, zb, 0)
        pltpu.sync_copy(zbuf_v, acc_sh.at[pl.ds(sid * ZSL, ZSL)])
        plsc.subcore_barrier()

        def blk(b, carry):
            row0 = wid * NCHUNK + b * KB
            pltpu.sync_copy(dst3_hbm.at[pl.ds(row0, KB)], idx_v)
            for k in range(KB):
                pltpu.sync_copy(ones_v, acc_sh.at[idx_v.at[k]], add=True)
            return carry

        lax.fori_loop(0, NCHUNK // KB, blk, 0)
        plsc.subcore_barrier()
        pltpu.sync_copy(acc_sh.at[pl.ds(sid * ZSL, ZSL)],
                        out_hbm.at[cid, pl.ds(sid * ZSL, ZSL)])

    return deg_k(dst3_2d)


SW = DB // NW

def _inv(deg0, deg1):
    @functools.partial(
        pl.kernel,
        out_type=jax.ShapeDtypeStruct((DB,), jnp.float32),
        mesh=_mesh(),
        scratch_types=[
            pltpu.VMEM((SW,), jnp.float32),
            pltpu.VMEM((SW,), jnp.float32),
            pltpu.VMEM((SW,), jnp.float32),
        ],
    )
    def inv_k(d0_hbm, d1_hbm, inv_hbm, d0_v, d1_v, o_v):
        cid = lax.axis_index("c")
        sid = lax.axis_index("s")
        wid = cid * NS + sid
        base = wid * SW
        pltpu.sync_copy(d0_hbm.at[pl.ds(base, SW)], d0_v)
        pltpu.sync_copy(d1_hbm.at[pl.ds(base, SW)], d1_v)

        def gg(i, carry):
            d = d0_v[pl.ds(i * 16, 16)] + d1_v[pl.ds(i * 16, 16)]
            inv = 1.0 / jnp.maximum(d, 1.0)
            b = base + i * 16 + lax.iota(jnp.int32, 16)
            o_v[pl.ds(i * 16, 16)] = jnp.where(b % NP < N, inv, 0.0)
            return carry

        lax.fori_loop(0, SW // 16, gg, 0)
        pltpu.sync_copy(o_v, inv_hbm.at[pl.ds(base, SW)])

    return inv_k(deg0, deg1)


def _agg_pass(srcR, dstR, y3, zz, d):
    @functools.partial(
        pl.kernel,
        out_type=jax.ShapeDtypeStruct((NC, R, NP, d), jnp.float32),
        mesh=_mesh(),
        scratch_types=[
            pltpu.VMEM((KB, CHUNK), jnp.int32),
            pltpu.VMEM((KB, CHUNK), jnp.int32),
            pltpu.VMEM((CHUNK, d), jnp.float32),
            pltpu.VMEM((CHUNK, d), jnp.float32),
            pltpu.VMEM_SHARED((NP, d), jnp.float32),
            pltpu.SemaphoreType.DMA,
            pltpu.SemaphoreType.DMA,
        ],
    )
    def e_k(srcR_hbm, dstR_hbm, y3_hbm, zz_hbm, out_hbm,
            sidx_v, didx_v, rows_a, rows_b, acc_sh, gsem_a, gsem_b):
        cid = lax.axis_index("c")
        sid = lax.axis_index("s")
        wid = cid * NS + sid
        bufs = (rows_a, rows_b)
        gsems = (gsem_a, gsem_b)

        pltpu.sync_copy(zz_hbm.at[pl.ds(sid * RPT, RPT)],
                        acc_sh.at[pl.ds(sid * RPT, RPT)])
        plsc.subcore_barrier()

        for r in range(R):
            def blk(b, carry):
                row0 = r * PR_ROWS + wid * NCHUNK_R + b * KB
                pltpu.sync_copy(srcR_hbm.at[pl.ds(row0, KB)], sidx_v)
                pltpu.sync_copy(dstR_hbm.at[pl.ds(row0, KB)], didx_v)
                gd = [None, None]
                gd[0] = pltpu.async_copy(y3_hbm.at[sidx_v.at[0]], bufs[0],
                                         gsems[0])
                for k in range(KB):
                    nk = k + 1
                    if nk < KB:
                        gd[nk % 2] = pltpu.async_copy(
                            y3_hbm.at[sidx_v.at[nk]], bufs[nk % 2],
                            gsems[nk % 2])
                    gd[k % 2].wait()
                    pltpu.sync_copy(bufs[k % 2], acc_sh.at[didx_v.at[k]],
                                    add=True)
                return carry

            lax.fori_loop(0, NCHUNK_R // KB, blk, 0)
            plsc.subcore_barrier()
            pltpu.sync_copy(acc_sh.at[pl.ds(sid * RPT, RPT)],
                            out_hbm.at[cid, r, pl.ds(sid * RPT, RPT)])
            if r + 1 < R:
                pltpu.sync_copy(zz_hbm.at[pl.ds(sid * RPT, RPT)],
                                acc_sh.at[pl.ds(sid * RPT, RPT)])
            plsc.subcore_barrier()

    return e_k(srcR, dstR, y3, zz)


def _mm(xin, W, lw, b, dout, fuse):
    BM = 512
    grid = (NP // BM,)

    def body_plain(x_ref, w_ref, lw_ref, b_ref, y3_ref, sl_ref):
        xb = x_ref[...]
        for r in range(R):
            y3_ref[r] = jnp.dot(xb, w_ref[r], preferred_element_type=jnp.float32)
        sl_ref[...] = (jnp.dot(xb, lw_ref[...], preferred_element_type=jnp.float32)
                       + b_ref[...])

    def body_fuse(a_ref, inv_ref, sl_ref_in, w_ref, lw_ref, b_ref, y3_ref, sl_ref):
        xb = sl_ref_in[...]
        for r in range(R):
            xb = xb + (a_ref[0, r] + a_ref[1, r]) * inv_ref[r]
        xb = jax.nn.relu(xb)
        for r in range(R):
            y3_ref[r] = jnp.dot(xb, w_ref[r], preferred_element_type=jnp.float32)
        sl_ref[...] = (jnp.dot(xb, lw_ref[...], preferred_element_type=jnp.float32)
                       + b_ref[...])

    din = 128
    w_specs = [
        pl.BlockSpec((R, din, dout), lambda i: (0, 0, 0)),
        pl.BlockSpec((din, dout), lambda i: (0, 0)),
        pl.BlockSpec((1, dout), lambda i: (0, 0)),
    ]
    out_specs = [
        pl.BlockSpec((R, BM, dout), lambda i: (0, i, 0)),
        pl.BlockSpec((BM, dout), lambda i: (i, 0)),
    ]
    out_shape = [
        jax.ShapeDtypeStruct((R, NP, dout), jnp.float32),
        jax.ShapeDtypeStruct((NP, dout), jnp.float32),
    ]
    if fuse:
        aggs, inv3, slp = xin
        in_specs = [
            pl.BlockSpec((NC, R, BM, din), lambda i: (0, 0, i, 0)),
            pl.BlockSpec((R, BM, 1), lambda i: (0, i, 0)),
            pl.BlockSpec((BM, din), lambda i: (i, 0)),
        ] + w_specs
        return pl.pallas_call(
            body_fuse, grid=grid, in_specs=in_specs, out_specs=out_specs,
            out_shape=out_shape)(aggs, inv3, slp, W, lw, b)
    in_specs = [pl.BlockSpec((BM, din), lambda i: (i, 0))] + w_specs
    return pl.pallas_call(
        body_plain, grid=grid, in_specs=in_specs, out_specs=out_specs,
        out_shape=out_shape)(xin, W, lw, b)


def _final_add(aggs, inv3, slp):
    BM = 1024

    def body(a_ref, inv_ref, sl_ref, o_ref):
        xb = sl_ref[...]
        for r in range(R):
            xb = xb + (a_ref[0, r] + a_ref[1, r]) * inv_ref[r]
        o_ref[...] = xb[:, :DOUT]

    return pl.pallas_call(
        body,
        grid=(NP // BM,),
        in_specs=[
            pl.BlockSpec((NC, R, BM, DH), lambda i: (0, 0, i, 0)),
            pl.BlockSpec((R, BM, 1), lambda i: (0, i, 0)),
            pl.BlockSpec((BM, DH), lambda i: (i, 0)),
        ],
        out_specs=pl.BlockSpec((BM, DOUT), lambda i: (i, 0)),
        out_shape=jax.ShapeDtypeStruct((NP, DOUT), jnp.float32),
    )(aggs, inv3, slp)


def kernel(x, edge_index_r0, edge_index_r1, edge_index_r2,
           W0, loopW0, loopb0, W1, loopW1, loopb1, W2, loopW2, loopb2):
    padR = PR_ROWS * CHUNK - E
    prr = jnp.arange(padR, dtype=jnp.int32)
    eis = (edge_index_r0, edge_index_r1, edge_index_r2)
    srcR = jnp.concatenate(
        [jnp.concatenate([ei[0] + r * NP, prr % N + r * NP])
         for r, ei in enumerate(eis)]).reshape(R * PR_ROWS, CHUNK).astype(jnp.int32)
    dstR = jnp.concatenate(
        [jnp.concatenate([ei[1], N + prr % (NP - N)])
         for ei in eis]).reshape(R * PR_ROWS, CHUNK).astype(jnp.int32)
    roff = jnp.repeat(jnp.arange(R, dtype=jnp.int32) * NP, PR_ROWS)[:, None]
    dst3_2d = dstR + roff

    degP = _deg(dst3_2d)
    inv3 = _inv(degP[0], degP[1]).reshape(R, NP, 1)

    zz_h = jnp.zeros((NP, DH), jnp.float32)

    x_p = jnp.pad(x, ((0, NP - N), (0, 0)))
    y3, sl = _mm(x_p, W0, loopW0, loopb0.reshape(1, -1), DH, fuse=False)
    y3, inv3 = lax.optimization_barrier((y3, inv3))
    aggs = _agg_pass(srcR, dstR, y3.reshape(R * NP, DH), zz_h, DH)

    y3, sl = _mm((aggs, inv3, sl), W1, loopW1, loopb1.reshape(1, -1), DH, fuse=True)
    aggs = _agg_pass(srcR, dstR, y3.reshape(R * NP, DH), zz_h, DH)

    W2p = jnp.pad(W2, ((0, 0), (0, 0), (0, DH - DOUT)))
    lw2p = jnp.pad(loopW2, ((0, 0), (0, DH - DOUT)))
    b2p = jnp.pad(loopb2, (0, DH - DOUT))
    y3, sl = _mm((aggs, inv3, sl), W2p, lw2p, b2p.reshape(1, -1), DH, fuse=True)
    aggs = _agg_pass(srcR, dstR, y3.reshape(R * NP, DH), zz_h, DH)

    return _final_add(aggs, inv3, sl)[:N]
